# baseline (speedup 1.0000x reference)
import functools

import jax
import jax.numpy as jnp
from jax import lax
from jax.experimental import pallas as pl
from jax.experimental.pallas import tpu as pltpu
from jax.experimental.pallas import tpu_sc as plsc

_LEAK = 0.01
_EB = 4096
_NB = 1024
_M = 12


def _leaky(x):
    return jnp.where(x > 0, x, _LEAK * x)


def _softplus(x):
    return jnp.log1p(jnp.exp(-jnp.abs(x))) + jnp.maximum(x, 0.0)


def _sigmoid(x):
    return 1.0 / (1.0 + jnp.exp(-x))


@functools.partial(jax.jit, static_argnames=("chunk_rows",))
def _sc_gather(table, idx, chunk_rows=768):
    V, D = table.shape
    B = idx.shape[0]
    info = plsc.get_sparse_core_info()
    NC, NS = info.num_cores, info.num_subcores
    NW = NC * NS
    b_per_w = B // NW
    nchunks = b_per_w // chunk_rows
    assert b_per_w % chunk_rows == 0 and B % (8 * NW) == 0

    mesh = plsc.VectorSubcoreMesh(core_axis_name="c", subcore_axis_name="s")

    @functools.partial(
        pl.kernel, mesh=mesh,
        out_type=jax.ShapeDtypeStruct((B, D), jnp.float32),
        scratch_types=[
            pltpu.VMEM((chunk_rows,), jnp.int32),
            pltpu.VMEM((chunk_rows, D), jnp.float32),
            pltpu.SemaphoreType.DMA,
        ],
    )
    def k(table_hbm, idx_hbm, out_hbm, idx_v, rows_v, sem):
        wid = lax.axis_index("s") * NC + lax.axis_index("c")
        base = wid * b_per_w
        for c in range(nchunks):
            off = base + c * chunk_rows
            pltpu.sync_copy(idx_hbm.at[pl.ds(off, chunk_rows)], idx_v)
            pltpu.async_copy(table_hbm.at[idx_v], rows_v, sem).wait()
            pltpu.sync_copy(rows_v, out_hbm.at[pl.ds(off, chunk_rows)])

    return k(table, idx)


def _linear_tc(x, W, b, rows_blk):
    N, Din = x.shape
    Dout = W.shape[1]
    grid = (N // rows_blk,)

    def body(x_ref, w_ref, b_ref, o_ref):
        o_ref[...] = x_ref[...] @ w_ref[...] + b_ref[...]

    return pl.pallas_call(
        body,
        grid=grid,
        in_specs=[
            pl.BlockSpec((rows_blk, Din), lambda i: (i, 0)),
            pl.BlockSpec((Din, Dout), lambda i: (0, 0)),
            pl.BlockSpec((1, Dout), lambda i: (0, 0)),
        ],
        out_specs=pl.BlockSpec((rows_blk, Dout), lambda i: (i, 0)),
        out_shape=jax.ShapeDtypeStruct((N, Dout), jnp.float32),
    )(x, W, b.reshape(1, Dout))


def _edge_mlp(es, en, ef, wgs, wgn, wge, bg1, wg2, bg2,
              wms, wmn, wme, bm1, wm2, bm2):
    E, Dt = es.shape
    De = ef.shape[1]
    H3 = wgs.shape[1]
    grid = (E // _EB,)

    def body(es_ref, en_ref, ef_ref, wgs_r, wgn_r, wge_r, bg1_r, wg2_r,
             bg2_r, wms_r, wmn_r, wme_r, bm1_r, wm2_r, bm2_r,
             gate_ref, msg_ref):
        esv = es_ref[...]
        env = en_ref[...]
        efv = ef_ref[...]
        h1 = _leaky(esv @ wgs_r[...] + env @ wgn_r[...] + efv @ wge_r[...]
                    + bg1_r[...])
        gate_ref[...] = h1 @ wg2_r[...] + bg2_r[...]
        h2 = _leaky(esv @ wms_r[...] + env @ wmn_r[...] + efv @ wme_r[...]
                    + bm1_r[...])
        msg_ref[...] = h2 @ wm2_r[...] + bm2_r[...]

    return pl.pallas_call(
        body,
        grid=grid,
        in_specs=[
            pl.BlockSpec((_EB, Dt), lambda i: (i, 0)),
            pl.BlockSpec((_EB, Dt), lambda i: (i, 0)),
            pl.BlockSpec((_EB, De), lambda i: (i, 0)),
            pl.BlockSpec((Dt, H3), lambda i: (0, 0)),
            pl.BlockSpec((Dt, H3), lambda i: (0, 0)),
            pl.BlockSpec((De, H3), lambda i: (0, 0)),
            pl.BlockSpec((1, H3), lambda i: (0, 0)),
            pl.BlockSpec((H3, 8), lambda i: (0, 0)),
            pl.BlockSpec((1, 8), lambda i: (0, 0)),
            pl.BlockSpec((Dt, H3), lambda i: (0, 0)),
            pl.BlockSpec((Dt, H3), lambda i: (0, 0)),
            pl.BlockSpec((De, H3), lambda i: (0, 0)),
            pl.BlockSpec((1, H3), lambda i: (0, 0)),
            pl.BlockSpec((H3, H3), lambda i: (0, 0)),
            pl.BlockSpec((1, H3), lambda i: (0, 0)),
        ],
        out_specs=[
            pl.BlockSpec((_EB, 8), lambda i: (i, 0)),
            pl.BlockSpec((_EB, H3), lambda i: (i, 0)),
        ],
        out_shape=[
            jax.ShapeDtypeStruct((E, 8), jnp.float32),
            jax.ShapeDtypeStruct((E, H3), jnp.float32),
        ],
    )(es, en, ef, wgs, wgn, wge, bg1, wg2, bg2, wms, wmn, wme, bm1, wm2, bm2)


def _node_mlp(x, wg1, bg1, wg2, bg2, wm1, bm1, wm2, bm2):
    N, D = x.shape
    H3 = wg1.shape[1]
    grid = (N // _EB,)

    def body(x_ref, wg1_r, bg1_r, wg2_r, bg2_r, wm1_r, bm1_r, wm2_r, bm2_r,
             gate_ref, msg_ref):
        xv = x_ref[...]
        h1 = _leaky(xv @ wg1_r[...] + bg1_r[...])
        gate_ref[...] = h1 @ wg2_r[...] + bg2_r[...]
        h2 = _leaky(xv @ wm1_r[...] + bm1_r[...])
        msg_ref[...] = h2 @ wm2_r[...] + bm2_r[...]

    return pl.pallas_call(
        body,
        grid=grid,
        in_specs=[
            pl.BlockSpec((_EB, D), lambda i: (i, 0)),
            pl.BlockSpec((D, H3), lambda i: (0, 0)),
            pl.BlockSpec((1, H3), lambda i: (0, 0)),
            pl.BlockSpec((H3, 8), lambda i: (0, 0)),
            pl.BlockSpec((1, 8), lambda i: (0, 0)),
            pl.BlockSpec((D, H3), lambda i: (0, 0)),
            pl.BlockSpec((1, H3), lambda i: (0, 0)),
            pl.BlockSpec((H3, H3), lambda i: (0, 0)),
            pl.BlockSpec((1, H3), lambda i: (0, 0)),
        ],
        out_specs=[
            pl.BlockSpec((_EB, 8), lambda i: (i, 0)),
            pl.BlockSpec((_EB, H3), lambda i: (i, 0)),
        ],
        out_shape=[
            jax.ShapeDtypeStruct((N, 8), jnp.float32),
            jax.ShapeDtypeStruct((N, H3), jnp.float32),
        ],
    )(x, wg1, bg1.reshape(1, -1), wg2, bg2, wm1, bm1.reshape(1, -1),
      wm2, bm2)


def _fc_bn_embed(x, w1, b1, g1, be1, w2, b2, wcg, bcg):
    N, D = x.shape

    def body(x_ref, w1_r, b1_r, g1_r, be1_r, w2_r, b2_r, wcg_r, bcg_r, o_ref):
        y = x_ref[...] @ w1_r[...] + b1_r[...]
        mu = jnp.mean(y, axis=0, keepdims=True)
        var = jnp.mean((y - mu) ** 2, axis=0, keepdims=True)
        z = g1_r[...] * (y - mu) / jnp.sqrt(var + 1e-5) + be1_r[...]
        z = _leaky(z)
        y2 = z @ w2_r[...] + b2_r[...]
        o_ref[...] = y2 @ wcg_r[...] + bcg_r[...]

    return pl.pallas_call(
        body,
        out_shape=jax.ShapeDtypeStruct((N, wcg.shape[1]), jnp.float32),
    )(x, w1, b1.reshape(1, -1), g1.reshape(1, -1), be1.reshape(1, -1),
      w2, b2.reshape(1, -1), wcg, bcg.reshape(1, -1))


def _conv_pre(atom, anbr3, bond3, ws, wn, wb, bias):
    N, D = atom.shape
    F = ws.shape[1]
    Db = bond3.shape[2]
    grid = (N // _NB,)
    nb = N // _NB

    Dn = anbr3.shape[2]

    def body(a_ref, an_ref, bd_ref, ws_r, wn_r, wb_r, b_r,
             g_ref, s_ref, q_ref):
        self_t = a_ref[...] @ ws_r[...]
        an2 = an_ref[...].reshape(_NB * _M, Dn)
        bd2 = bd_ref[...].reshape(_NB * _M, Db)
        g2 = an2 @ wn_r[...] + bd2 @ wb_r[...] + b_r[...]
        g3 = g2.reshape(_NB, _M, F) + self_t[:, None, :]
        g_ref[...] = g3
        gf = g3.reshape(_NB * _M, F)
        s_ref[...] = jnp.sum(gf, axis=0).reshape(1, 1, F)
        q_ref[...] = jnp.sum(gf * gf, axis=0).reshape(1, 1, F)

    return pl.pallas_call(
        body,
        grid=grid,
        in_specs=[
            pl.BlockSpec((_NB, D), lambda i: (i, 0)),
            pl.BlockSpec((_NB, _M, Dn), lambda i: (i, 0, 0)),
            pl.BlockSpec((_NB, _M, Db), lambda i: (i, 0, 0)),
            pl.BlockSpec((D, F), lambda i: (0, 0)),
            pl.BlockSpec((Dn, F), lambda i: (0, 0)),
            pl.BlockSpec((Db, F), lambda i: (0, 0)),
            pl.BlockSpec((1, F), lambda i: (0, 0)),
        ],
        out_specs=[
            pl.BlockSpec((_NB, _M, F), lambda i: (i, 0, 0)),
            pl.BlockSpec((1, 1, F), lambda i: (i, 0, 0)),
            pl.BlockSpec((1, 1, F), lambda i: (i, 0, 0)),
        ],
        out_shape=[
            jax.ShapeDtypeStruct((N, _M, F), jnp.float32),
            jax.ShapeDtypeStruct((nb, 1, F), jnp.float32),
            jax.ShapeDtypeStruct((nb, 1, F), jnp.float32),
        ],
    )(atom, anbr3, bond3, ws, wn, wb, bias)


def _conv_apply(g, stats):
    N = g.shape[0]
    F = g.shape[2]
    D = F // 2
    grid = (N // _NB,)
    nb = N // _NB

    def body(g_ref, st_r, s_ref, ps_ref, pq_ref):
        mu = st_r[0:1, :]
        rstd = st_r[1:2, :]
        ga = st_r[2:3, :]
        be = st_r[3:4, :]
        acc = jnp.zeros((_NB, D), jnp.float32)
        for m in range(_M):
            gm = g_ref[:, m, :]
            gn = ga * (gm - mu) * rstd + be
            filt = gn[:, :D]
            core = gn[:, D:]
            acc = acc + _sigmoid(filt) * _softplus(core)
        s_ref[...] = acc
        ps_ref[...] = jnp.sum(acc, axis=0).reshape(1, 1, D)
        pq_ref[...] = jnp.sum(acc * acc, axis=0).reshape(1, 1, D)

    return pl.pallas_call(
        body,
        grid=grid,
        in_specs=[
            pl.BlockSpec((_NB, _M, F), lambda i: (i, 0, 0)),
            pl.BlockSpec((8, F), lambda i: (0, 0)),
        ],
        out_specs=[
            pl.BlockSpec((_NB, D), lambda i: (i, 0)),
            pl.BlockSpec((1, 1, D), lambda i: (i, 0, 0)),
            pl.BlockSpec((1, 1, D), lambda i: (i, 0, 0)),
        ],
        out_shape=[
            jax.ShapeDtypeStruct((N, D), jnp.float32),
            jax.ShapeDtypeStruct((nb, 1, D), jnp.float32),
            jax.ShapeDtypeStruct((nb, 1, D), jnp.float32),
        ],
    )(g, stats)


def _conv_post(atom, s, stats2):
    N, D = atom.shape

    def body(a_ref, s_ref, st_r, o_ref):
        mu = st_r[0:1, :]
        rstd = st_r[1:2, :]
        ga = st_r[2:3, :]
        be = st_r[3:4, :]
        z = ga * (s_ref[...] - mu) * rstd + be
        o_ref[...] = _softplus(a_ref[...] + z)

    return pl.pallas_call(
        body,
        out_shape=jax.ShapeDtypeStruct((N, D), jnp.float32),
    )(atom, s, stats2)


def _out_head(crys, wcf, bcf, wo, bo):
    Bc, D = crys.shape
    H = wcf.shape[1]

    def body(c_ref, wcf_r, bcf_r, wo_r, bo_r, o_ref):
        c1 = _softplus(c_ref[...])
        h = _softplus(c1 @ wcf_r[...] + bcf_r[...])
        o_ref[...] = h @ wo_r[...] + bo_r[...]

    return pl.pallas_call(
        body,
        out_shape=jax.ShapeDtypeStruct((Bc, 8), jnp.float32),
    )(crys, wcf, bcf.reshape(1, -1), wo, bo)


def _pack_heads(layer, din_s, din_n, din_e, dtab):
    H = 64
    nh = len(layer)
    wg1 = jnp.concatenate([hp["gate"]["hidden"][0][0] for hp in layer], 1)
    bg1 = jnp.concatenate([hp["gate"]["hidden"][0][1] for hp in layer])
    wm1 = jnp.concatenate([hp["msg"]["hidden"][0][0] for hp in layer], 1)
    bm1 = jnp.concatenate([hp["msg"]["hidden"][0][1] for hp in layer])

    def split_pad(w):
        ws = jnp.zeros((dtab, H * nh)).at[:din_s].set(w[:din_s])
        wn = jnp.zeros((dtab, H * nh)).at[:din_n].set(w[din_s:din_s + din_n])
        we = jnp.zeros((8, H * nh)).at[:din_e].set(w[din_s + din_n:])
        return ws, wn, we

    wgs, wgn, wge = split_pad(wg1)
    wms, wmn, wme = split_pad(wm1)

    wg2 = jnp.zeros((H * nh, 8))
    bg2 = jnp.zeros((1, 8))
    wm2 = jnp.zeros((H * nh, H * nh))
    bm2 = jnp.zeros((1, H * nh))
    for h, hp in enumerate(layer):
        wg2 = wg2.at[h * H:(h + 1) * H, h].set(hp["gate"]["out"][0][:, 0])
        bg2 = bg2.at[0, h].set(hp["gate"]["out"][1][0])
        wm2 = wm2.at[h * H:(h + 1) * H, h * H:(h + 1) * H].set(
            hp["msg"]["out"][0])
        bm2 = bm2.at[0, h * H:(h + 1) * H].set(hp["msg"]["out"][1])
    pows = jnp.stack([hp["pow"][0] for hp in layer])
    return (wgs, wgn, wge, bg1.reshape(1, -1), wg2, bg2,
            wms, wmn, wme, bm1.reshape(1, -1), wm2, bm2, pows)


def _pack_node_heads(heads):
    H = 64
    nh = len(heads)
    wg1 = jnp.concatenate([hp["gate"]["hidden"][0][0] for hp in heads], 1)
    bg1 = jnp.concatenate([hp["gate"]["hidden"][0][1] for hp in heads])
    wm1 = jnp.concatenate([hp["msg"]["hidden"][0][0] for hp in heads], 1)
    bm1 = jnp.concatenate([hp["msg"]["hidden"][0][1] for hp in heads])
    wg2 = jnp.zeros((H * nh, 8))
    bg2 = jnp.zeros((1, 8))
    wm2 = jnp.zeros((H * nh, H * nh))
    bm2 = jnp.zeros((1, H * nh))
    for h, hp in enumerate(heads):
        wg2 = wg2.at[h * H:(h + 1) * H, h].set(hp["gate"]["out"][0][:, 0])
        bg2 = bg2.at[0, h].set(hp["gate"]["out"][1][0])
        wm2 = wm2.at[h * H:(h + 1) * H, h * H:(h + 1) * H].set(
            hp["msg"]["out"][0])
        bm2 = bm2.at[0, h * H:(h + 1) * H].set(hp["msg"]["out"][1])
    pows = jnp.stack([hp["pow"][0] for hp in heads])
    return wg1, bg1, wg2, bg2, wm1, bm1, wm2, bm2, pows


def _bn_stats(sums, sqs, count, gamma, beta, width):
    mu = sums.reshape(-1, sums.shape[-1]).sum(0) / count
    ex2 = sqs.reshape(-1, sqs.shape[-1]).sum(0) / count
    var = ex2 - mu * mu
    rstd = 1.0 / jnp.sqrt(var + 1e-5)
    st = jnp.zeros((8, width))
    st = st.at[0].set(mu).at[1].set(rstd).at[2].set(gamma).at[3].set(beta)
    return st


def _segment_softmax_apply(gates, msgs, idx, weights_g, pows, num_segments):
    outs = []
    gs = []
    E = gates.shape[0]
    for h in range(pows.shape[0]):
        g = gates[:, h:h + 1]
        smax = jax.ops.segment_max(g, idx, num_segments=num_segments)
        smax = jnp.where(jnp.isfinite(smax), smax, 0.0)
        g = g - smax[idx]
        g = (weights_g ** pows[h]) * jnp.exp(g)
        denom = jax.ops.segment_sum(g, idx, num_segments=num_segments)
        g = g / (denom[idx] + 1e-10)
        out = jax.ops.segment_sum(g * msgs[:, h * 64:(h + 1) * 64], idx,
                                  num_segments=num_segments)
        outs.append(out)
        gs.append(g)
    return outs, gs


def kernel(comp_weights, comp_fea, edge_fea, self_fea_idx, comp_nbr_fea_idx,
           comp_node_idx, struct_nbr_fea, struct_nbr_fea_idx,
           struct_node_idx, params):
    N_COMP = comp_fea.shape[0]
    E = self_fea_idx.shape[0]
    N_STRUCT = struct_nbr_fea.shape[0]
    BOND = struct_nbr_fea.shape[2]
    Bc = 1024

    W, b = params["embed"]
    elem = _linear_tc(comp_fea, W, b, 4096)

    ef8 = jnp.zeros((E, 8), jnp.float32).at[:, :2].set(edge_fea)
    idx_self = self_fea_idx.astype(jnp.int32)
    idx_nbr = comp_nbr_fea_idx.astype(jnp.int32)

    gate_list = []
    for layer in params["mp"]:
        packed = _pack_heads(layer, 64, 64, 2, 128)
        (wgs, wgn, wge, bg1, wg2, bg2,
         wms, wmn, wme, bm1, wm2, bm2, pows) = packed
        table = jnp.concatenate(
            [elem, comp_weights, jnp.zeros((N_COMP, 63), jnp.float32)], 1)
        es = _sc_gather(table, idx_self)
        en = _sc_gather(table, idx_nbr)
        nw = en[:, 64:65]
        gates, msgs = _edge_mlp(es, en, ef8, wgs, wgn, wge, bg1, wg2, bg2,
                                wms, wmn, wme, bm1, wm2, bm2)
        outs, gs = _segment_softmax_apply(gates, msgs, idx_self, nw, pows,
                                          N_COMP)
        elem = elem + (outs[0] + outs[1] + outs[2]) / 3.0
        gate_list.append((gs[0] + gs[1] + gs[2]) / 3.0)

    wg1, bg1, wg2, bg2, wm1, bm1, wm2, bm2, pows = _pack_node_heads(
        params["cry"])
    gates, msgs = _node_mlp(elem, wg1, bg1, wg2, bg2, wm1, bm1, wm2, bm2)
    outs, _ = _segment_softmax_apply(gates, msgs, comp_node_idx,
                                     comp_weights, pows, N_STRUCT)
    node_fea = (outs[0] + outs[1] + outs[2]) / 3.0

    (w1, b1), = params["fc"]["hidden"]
    g1, be1 = params["fc"]["bn"][0]
    w2, b2 = params["fc"]["out"]
    wcg, bcg = params["cg_embed"]
    atom = _fc_bn_embed(node_fea, w1, b1, g1, be1, w2, b2, wcg, bcg)

    nbr_flat = struct_nbr_fea_idx.reshape(-1).astype(jnp.int32)
    bond3 = jnp.zeros((N_STRUCT, _M, 48), jnp.float32).at[:, :, :BOND].set(
        struct_nbr_fea)
    for p in params["conv"]:
        wf, bf = p["fc_full"]
        ws = wf[:64]
        wn = jnp.zeros((128, wf.shape[1]), jnp.float32).at[:64].set(
            wf[64:128])
        wb = jnp.zeros((48, wf.shape[1]), jnp.float32).at[:BOND].set(wf[128:])
        atom_pad = jnp.concatenate(
            [atom, jnp.zeros((N_STRUCT, 64), jnp.float32)], 1)
        anbr = _sc_gather(atom_pad, nbr_flat)
        anbr3 = anbr.reshape(N_STRUCT, _M, 128)
        g, s1, q1 = _conv_pre(atom, anbr3, bond3, ws, wn, wb,
                              bf.reshape(1, -1))
        ga1, be1c = p["bn1"]
        st1 = _bn_stats(s1, q1, N_STRUCT * _M, ga1, be1c, wf.shape[1])
        ssum, s2, q2 = _conv_apply(g, st1)
        ga2, be2c = p["bn2"]
        st2 = _bn_stats(s2, q2, N_STRUCT, ga2, be2c, 64)
        atom = _conv_post(atom, ssum, st2)

    ones = jnp.ones((N_STRUCT, 1), jnp.float32)
    counts = jax.ops.segment_sum(ones, struct_node_idx, num_segments=Bc)
    crys = jax.ops.segment_sum(atom, struct_node_idx, num_segments=Bc)
    crys = crys / jnp.maximum(counts, 1.0)
    wcf, bcf = params["conv_to_fc"]
    wo, bo = params["fc_out"]
    wo8 = jnp.zeros((wcf.shape[1], 8), jnp.float32).at[:, :1].set(wo)
    bo8 = jnp.zeros((1, 8), jnp.float32).at[0, 0].set(bo[0])
    out = _out_head(crys, wcf, bcf, wo8, bo8)[:, :1]

    return (out,) + tuple(gate_list)
